# initial kernel scaffold (unmeasured)
import jax
import jax.numpy as jnp
from jax import lax
from jax.experimental import pallas as pl
from jax.experimental.pallas import tpu as pltpu

N_DEV = 4


def kernel(x, w_mat):
    M, K = x.shape
    _, N = w_mat.shape
    Mc = M // N_DEV

    def body(
        x_ref, w_ref, out_ref,
        wbf_ref, send_ref, recv_ref,
        amax_send_ref, amax_recv_ref,
        send_sems, recv_sems, amax_send_sems, amax_recv_sems,
    ):
        my = lax.axis_index("i")
        wbf_ref[...] = w_ref[...].astype(jnp.bfloat16)

        for t in range(1, N_DEV):
            tgt = lax.rem(my + t, N_DEV)
            xc = x_ref[pl.ds(tgt * Mc, Mc), :].astype(jnp.bfloat16)
            send_ref[t - 1, :, :] = jnp.dot(
                xc, wbf_ref[...], preferred_element_type=jnp.float32
            ).astype(jnp.bfloat16)
            pltpu.make_async_remote_copy(
                src_ref=send_ref.at[t - 1],
                dst_ref=recv_ref.at[t - 1],
                send_sem=send_sems.at[t - 1],
                recv_sem=recv_sems.at[t - 1],
                device_id=(tgt,),
                device_id_type=pl.DeviceIdType.MESH,
            ).start()

        xo = x_ref[pl.ds(my * Mc, Mc), :].astype(jnp.bfloat16)
        out_ref[...] = jnp.dot(
            xo, wbf_ref[...], preferred_element_type=jnp.float32
        )

        for j in range(N_DEV - 1):
            pltpu.make_async_remote_copy(
                src_ref=send_ref.at[j],
                dst_ref=recv_ref.at[j],
                send_sem=send_sems.at[j],
                recv_sem=recv_sems.at[j],
                device_id=(my,),
                device_id_type=pl.DeviceIdType.MESH,
            ).wait_recv()
            out_ref[...] = out_ref[...] + recv_ref[j].astype(jnp.float32)

        amax = jnp.max(jnp.abs(out_ref[...]))
        amax_send_ref[...] = jnp.full((8, 128), amax, jnp.float32)
        for t in range(1, N_DEV):
            tgt = lax.rem(my + t, N_DEV)
            pltpu.make_async_remote_copy(
                src_ref=amax_send_ref,
                dst_ref=amax_recv_ref.at[t - 1],
                send_sem=amax_send_sems.at[t - 1],
                recv_sem=amax_recv_sems.at[t - 1],
                device_id=(tgt,),
                device_id_type=pl.DeviceIdType.MESH,
            ).start()
        for j in range(N_DEV - 1):
            pltpu.make_async_remote_copy(
                src_ref=amax_send_ref,
                dst_ref=amax_recv_ref.at[j],
                send_sem=amax_send_sems.at[j],
                recv_sem=amax_recv_sems.at[j],
                device_id=(my,),
                device_id_type=pl.DeviceIdType.MESH,
            ).wait_recv()
            amax = jnp.maximum(amax, amax_recv_ref[j, 0, 0])

        scale = amax / 448.0
        q = jnp.clip(out_ref[...] / scale, -448.0, 448.0).astype(
            jnp.float8_e4m3fn
        )
        out_ref[...] = q.astype(jnp.float32) * scale

        for j in range(N_DEV - 1):
            pltpu.make_async_remote_copy(
                src_ref=send_ref.at[j],
                dst_ref=recv_ref.at[j],
                send_sem=send_sems.at[j],
                recv_sem=recv_sems.at[j],
                device_id=(my,),
                device_id_type=pl.DeviceIdType.MESH,
            ).wait_send()
            pltpu.make_async_remote_copy(
                src_ref=amax_send_ref,
                dst_ref=amax_recv_ref.at[j],
                send_sem=amax_send_sems.at[j],
                recv_sem=amax_recv_sems.at[j],
                device_id=(my,),
                device_id_type=pl.DeviceIdType.MESH,
            ).wait_send()

    return pl.pallas_call(
        body,
        out_shape=jax.ShapeDtypeStruct((Mc, N), jnp.float32),
        in_specs=[
            pl.BlockSpec(memory_space=pltpu.VMEM),
            pl.BlockSpec(memory_space=pltpu.VMEM),
        ],
        out_specs=pl.BlockSpec(memory_space=pltpu.VMEM),
        scratch_shapes=[
            pltpu.VMEM((K, N), jnp.bfloat16),
            pltpu.VMEM((N_DEV - 1, Mc, N), jnp.bfloat16),
            pltpu.VMEM((N_DEV - 1, Mc, N), jnp.bfloat16),
            pltpu.VMEM((8, 128), jnp.float32),
            pltpu.VMEM((N_DEV - 1, 8, 128), jnp.float32),
            pltpu.SemaphoreType.DMA((N_DEV - 1,)),
            pltpu.SemaphoreType.DMA((N_DEV - 1,)),
            pltpu.SemaphoreType.DMA((N_DEV - 1,)),
            pltpu.SemaphoreType.DMA((N_DEV - 1,)),
        ],
    )(x, w_mat)


# baseline (device time: 142028 ns/iter reference)
import jax
import jax.numpy as jnp
from jax import lax
from jax.experimental import pallas as pl
from jax.experimental.pallas import tpu as pltpu

N_DEV = 4


def kernel(x, w_mat):
    M, K = x.shape
    _, N = w_mat.shape
    Mc = M // N_DEV

    def body(
        x_ref, w_ref, out_ref,
        send_ref, recv_ref,
        amax_send_ref, amax_recv_ref,
        send_sems, recv_sems, amax_send_sems, amax_recv_sems,
    ):
        my = lax.axis_index("i")

        for t in range(1, N_DEV):
            tgt = lax.rem(my + t, N_DEV)
            xc = x_ref[pl.ds(tgt * Mc, Mc), :]
            send_ref[t - 1, :, :] = jnp.dot(
                xc, w_ref[...], preferred_element_type=jnp.float32
            ).astype(jnp.bfloat16)
            pltpu.make_async_remote_copy(
                src_ref=send_ref.at[t - 1],
                dst_ref=recv_ref.at[t - 1],
                send_sem=send_sems.at[t - 1],
                recv_sem=recv_sems.at[t - 1],
                device_id=(tgt,),
                device_id_type=pl.DeviceIdType.MESH,
            ).start()

        xo = x_ref[pl.ds(my * Mc, Mc), :]
        out_ref[...] = jnp.dot(
            xo, w_ref[...], preferred_element_type=jnp.float32
        )

        for j in range(N_DEV - 1):
            pltpu.make_async_remote_copy(
                src_ref=send_ref.at[j],
                dst_ref=recv_ref.at[j],
                send_sem=send_sems.at[j],
                recv_sem=recv_sems.at[j],
                device_id=(my,),
                device_id_type=pl.DeviceIdType.MESH,
            ).wait_recv()
            out_ref[...] = out_ref[...] + recv_ref[j].astype(jnp.float32)

        amax = jnp.max(jnp.abs(out_ref[...]))
        amax_send_ref[...] = jnp.full((8, 128), amax, jnp.float32)
        for t in range(1, N_DEV):
            tgt = lax.rem(my + t, N_DEV)
            pltpu.make_async_remote_copy(
                src_ref=amax_send_ref,
                dst_ref=amax_recv_ref.at[t - 1],
                send_sem=amax_send_sems.at[t - 1],
                recv_sem=amax_recv_sems.at[t - 1],
                device_id=(tgt,),
                device_id_type=pl.DeviceIdType.MESH,
            ).start()
        for j in range(N_DEV - 1):
            pltpu.make_async_remote_copy(
                src_ref=amax_send_ref,
                dst_ref=amax_recv_ref.at[j],
                send_sem=amax_send_sems.at[j],
                recv_sem=amax_recv_sems.at[j],
                device_id=(my,),
                device_id_type=pl.DeviceIdType.MESH,
            ).wait_recv()
            amax = jnp.maximum(amax, amax_recv_ref[j, 0, 0])

        scale = amax / 448.0
        q = jnp.clip(out_ref[...] / scale, -448.0, 448.0).astype(
            jnp.float8_e4m3fn
        )
        out_ref[...] = q.astype(jnp.float32) * scale

        for j in range(N_DEV - 1):
            pltpu.make_async_remote_copy(
                src_ref=send_ref.at[j],
                dst_ref=recv_ref.at[j],
                send_sem=send_sems.at[j],
                recv_sem=recv_sems.at[j],
                device_id=(my,),
                device_id_type=pl.DeviceIdType.MESH,
            ).wait_send()
            pltpu.make_async_remote_copy(
                src_ref=amax_send_ref,
                dst_ref=amax_recv_ref.at[j],
                send_sem=amax_send_sems.at[j],
                recv_sem=amax_recv_sems.at[j],
                device_id=(my,),
                device_id_type=pl.DeviceIdType.MESH,
            ).wait_send()

    return pl.pallas_call(
        body,
        out_shape=jax.ShapeDtypeStruct((Mc, N), jnp.float32),
        in_specs=[
            pl.BlockSpec(memory_space=pltpu.VMEM),
            pl.BlockSpec(memory_space=pltpu.VMEM),
        ],
        out_specs=pl.BlockSpec(memory_space=pltpu.VMEM),
        scratch_shapes=[
            pltpu.VMEM((N_DEV - 1, Mc, N), jnp.bfloat16),
            pltpu.VMEM((N_DEV - 1, Mc, N), jnp.bfloat16),
            pltpu.VMEM((8, 128), jnp.float32),
            pltpu.VMEM((N_DEV - 1, 8, 128), jnp.float32),
            pltpu.SemaphoreType.DMA((N_DEV - 1,)),
            pltpu.SemaphoreType.DMA((N_DEV - 1,)),
            pltpu.SemaphoreType.DMA((N_DEV - 1,)),
            pltpu.SemaphoreType.DMA((N_DEV - 1,)),
        ],
        compiler_params=pltpu.CompilerParams(
            vmem_limit_bytes=100 * 1024 * 1024,
        ),
    )(x.astype(jnp.bfloat16), w_mat.astype(jnp.bfloat16))


# device time: 123246 ns/iter; 1.1524x vs baseline; 1.1524x over previous
import jax
import jax.numpy as jnp
from jax import lax
from jax.experimental import pallas as pl
from jax.experimental.pallas import tpu as pltpu

N_DEV = 4


def kernel(x, w_mat):
    M, K = x.shape
    _, N = w_mat.shape
    Mc = M // N_DEV

    def body(
        x_ref, w_ref, out_ref,
        send_nbr_ref, recv_nbr_ref,
        send_dq_ref, send_ds_ref, recv_dq_ref, recv_ds_ref,
        amax_send_ref, amax_recv_ref,
        nbr_send_sems, nbr_recv_sems, diag_send_sems, diag_recv_sems,
        amax_send_sems, amax_recv_sems,
    ):
        my = lax.axis_index("i")

        def dot_chunk(c):
            return jnp.dot(
                x_ref[pl.ds(c * Mc, Mc), :], w_ref[...],
                preferred_element_type=jnp.float32,
            )

        tgt = lax.rem(my + 1, N_DEV)
        send_nbr_ref[0, :, :] = dot_chunk(tgt).astype(jnp.bfloat16)
        pltpu.make_async_remote_copy(
            src_ref=send_nbr_ref.at[0],
            dst_ref=recv_nbr_ref.at[0],
            send_sem=nbr_send_sems.at[0],
            recv_sem=nbr_recv_sems.at[0],
            device_id=(tgt,),
            device_id_type=pl.DeviceIdType.MESH,
        ).start()

        tgt = lax.rem(my + 2, N_DEV)
        p = dot_chunk(tgt)
        s = (jnp.max(jnp.abs(p), axis=1, keepdims=True) / 127.0).astype(
            jnp.bfloat16
        )
        send_ds_ref[...] = jnp.broadcast_to(s, (Mc, 128))
        send_dq_ref[...] = jnp.clip(
            jnp.round(p / s.astype(jnp.float32)), -127.0, 127.0
        ).astype(jnp.int8)
        pltpu.make_async_remote_copy(
            src_ref=send_dq_ref,
            dst_ref=recv_dq_ref,
            send_sem=diag_send_sems.at[0],
            recv_sem=diag_recv_sems.at[0],
            device_id=(tgt,),
            device_id_type=pl.DeviceIdType.MESH,
        ).start()
        pltpu.make_async_remote_copy(
            src_ref=send_ds_ref,
            dst_ref=recv_ds_ref,
            send_sem=diag_send_sems.at[1],
            recv_sem=diag_recv_sems.at[1],
            device_id=(tgt,),
            device_id_type=pl.DeviceIdType.MESH,
        ).start()

        tgt = lax.rem(my + 3, N_DEV)
        send_nbr_ref[1, :, :] = dot_chunk(tgt).astype(jnp.bfloat16)
        pltpu.make_async_remote_copy(
            src_ref=send_nbr_ref.at[1],
            dst_ref=recv_nbr_ref.at[1],
            send_sem=nbr_send_sems.at[1],
            recv_sem=nbr_recv_sems.at[1],
            device_id=(tgt,),
            device_id_type=pl.DeviceIdType.MESH,
        ).start()

        out_ref[...] = dot_chunk(my)

        def wait_recv(dst, sem):
            pltpu.make_async_remote_copy(
                src_ref=dst, dst_ref=dst, send_sem=sem, recv_sem=sem,
                device_id=(my,), device_id_type=pl.DeviceIdType.MESH,
            ).wait_recv()

        wait_recv(recv_nbr_ref.at[0], nbr_recv_sems.at[0])
        out_ref[...] = out_ref[...] + recv_nbr_ref[0].astype(jnp.float32)

        wait_recv(recv_dq_ref, diag_recv_sems.at[0])
        wait_recv(recv_ds_ref, diag_recv_sems.at[1])
        out_ref[...] = out_ref[...] + (
            recv_dq_ref[...].astype(jnp.float32)
            * recv_ds_ref[:, 0:1].astype(jnp.float32)
        )

        wait_recv(recv_nbr_ref.at[1], nbr_recv_sems.at[1])
        out_ref[...] = out_ref[...] + recv_nbr_ref[1].astype(jnp.float32)

        amax = jnp.max(jnp.abs(out_ref[...]))
        amax_send_ref[...] = jnp.full((8, 128), amax, jnp.float32)
        for t in range(1, N_DEV):
            tgt = lax.rem(my + t, N_DEV)
            pltpu.make_async_remote_copy(
                src_ref=amax_send_ref,
                dst_ref=amax_recv_ref.at[t - 1],
                send_sem=amax_send_sems.at[t - 1],
                recv_sem=amax_recv_sems.at[t - 1],
                device_id=(tgt,),
                device_id_type=pl.DeviceIdType.MESH,
            ).start()
        for j in range(N_DEV - 1):
            wait_recv(amax_recv_ref.at[j], amax_recv_sems.at[j])
            amax = jnp.maximum(amax, amax_recv_ref[j, 0, 0])

        scale = amax / 448.0
        q = jnp.clip(out_ref[...] / scale, -448.0, 448.0).astype(
            jnp.float8_e4m3fn
        )
        out_ref[...] = q.astype(jnp.float32) * scale

        def wait_send(src, sem):
            pltpu.make_async_remote_copy(
                src_ref=src, dst_ref=src, send_sem=sem, recv_sem=sem,
                device_id=(my,), device_id_type=pl.DeviceIdType.MESH,
            ).wait_send()

        wait_send(send_nbr_ref.at[0], nbr_send_sems.at[0])
        wait_send(send_nbr_ref.at[1], nbr_send_sems.at[1])
        wait_send(send_dq_ref, diag_send_sems.at[0])
        wait_send(send_ds_ref, diag_send_sems.at[1])
        for j in range(N_DEV - 1):
            wait_send(amax_send_ref, amax_send_sems.at[j])

    return pl.pallas_call(
        body,
        out_shape=jax.ShapeDtypeStruct((Mc, N), jnp.float32),
        in_specs=[
            pl.BlockSpec(memory_space=pltpu.VMEM),
            pl.BlockSpec(memory_space=pltpu.VMEM),
        ],
        out_specs=pl.BlockSpec(memory_space=pltpu.VMEM),
        scratch_shapes=[
            pltpu.VMEM((2, Mc, N), jnp.bfloat16),
            pltpu.VMEM((2, Mc, N), jnp.bfloat16),
            pltpu.VMEM((Mc, N), jnp.int8),
            pltpu.VMEM((Mc, 128), jnp.bfloat16),
            pltpu.VMEM((Mc, N), jnp.int8),
            pltpu.VMEM((Mc, 128), jnp.bfloat16),
            pltpu.VMEM((8, 128), jnp.float32),
            pltpu.VMEM((N_DEV - 1, 8, 128), jnp.float32),
            pltpu.SemaphoreType.DMA((2,)),
            pltpu.SemaphoreType.DMA((2,)),
            pltpu.SemaphoreType.DMA((2,)),
            pltpu.SemaphoreType.DMA((2,)),
            pltpu.SemaphoreType.DMA((N_DEV - 1,)),
            pltpu.SemaphoreType.DMA((N_DEV - 1,)),
        ],
        compiler_params=pltpu.CompilerParams(
            vmem_limit_bytes=100 * 1024 * 1024,
        ),
    )(x.astype(jnp.bfloat16), w_mat.astype(jnp.bfloat16))


# device time: 113595 ns/iter; 1.2503x vs baseline; 1.0850x over previous
import jax
import jax.numpy as jnp
from jax import lax
from jax.experimental import pallas as pl
from jax.experimental.pallas import tpu as pltpu

N_DEV = 4


def kernel(x, w_mat):
    M, K = x.shape
    _, N = w_mat.shape
    Mc = M // N_DEV

    def body(
        x_ref, w_ref, out_ref,
        xbuf_ref,
        send_nbr_ref, recv_nbr_ref,
        send_dq_ref, send_ds_ref, recv_dq_ref, recv_ds_ref,
        amax_send_ref, amax_recv_ref,
        xcopy_sems,
        nbr_send_sems, nbr_recv_sems, diag_send_sems, diag_recv_sems,
        amax_send_sems, amax_recv_sems,
    ):
        my = lax.axis_index("i")

        def x_copy(c, slot):
            return pltpu.make_async_copy(
                x_ref.at[pl.ds(c * Mc, Mc), :],
                xbuf_ref.at[slot],
                xcopy_sems.at[slot],
            )

        chunks = [lax.rem(my + t, N_DEV) for t in (1, 2, 3)] + [my]
        x_copy(chunks[0], 0).start()
        x_copy(chunks[1], 1).start()

        def dot_chunk(k):
            slot = k % 2
            x_copy(chunks[k], slot).wait()
            p = jnp.dot(
                xbuf_ref[slot].astype(jnp.bfloat16), w_ref[...],
                preferred_element_type=jnp.float32,
            )
            if k + 2 < N_DEV:
                x_copy(chunks[k + 2], slot).start()
            return p

        send_nbr_ref[0, :, :] = dot_chunk(0).astype(jnp.bfloat16)
        pltpu.make_async_remote_copy(
            src_ref=send_nbr_ref.at[0],
            dst_ref=recv_nbr_ref.at[0],
            send_sem=nbr_send_sems.at[0],
            recv_sem=nbr_recv_sems.at[0],
            device_id=(chunks[0],),
            device_id_type=pl.DeviceIdType.MESH,
        ).start()

        p = dot_chunk(1)
        s = (jnp.max(jnp.abs(p), axis=1, keepdims=True) / 127.0).astype(
            jnp.bfloat16
        )
        send_ds_ref[...] = jnp.broadcast_to(s, (Mc, 128))
        send_dq_ref[...] = jnp.clip(
            jnp.round(p / s.astype(jnp.float32)), -127.0, 127.0
        ).astype(jnp.int8)
        pltpu.make_async_remote_copy(
            src_ref=send_dq_ref,
            dst_ref=recv_dq_ref,
            send_sem=diag_send_sems.at[0],
            recv_sem=diag_recv_sems.at[0],
            device_id=(chunks[1],),
            device_id_type=pl.DeviceIdType.MESH,
        ).start()
        pltpu.make_async_remote_copy(
            src_ref=send_ds_ref,
            dst_ref=recv_ds_ref,
            send_sem=diag_send_sems.at[1],
            recv_sem=diag_recv_sems.at[1],
            device_id=(chunks[1],),
            device_id_type=pl.DeviceIdType.MESH,
        ).start()

        send_nbr_ref[1, :, :] = dot_chunk(2).astype(jnp.bfloat16)
        pltpu.make_async_remote_copy(
            src_ref=send_nbr_ref.at[1],
            dst_ref=recv_nbr_ref.at[1],
            send_sem=nbr_send_sems.at[1],
            recv_sem=nbr_recv_sems.at[1],
            device_id=(chunks[2],),
            device_id_type=pl.DeviceIdType.MESH,
        ).start()

        out_ref[...] = dot_chunk(3)

        def wait_recv(dst, sem):
            pltpu.make_async_remote_copy(
                src_ref=dst, dst_ref=dst, send_sem=sem, recv_sem=sem,
                device_id=(my,), device_id_type=pl.DeviceIdType.MESH,
            ).wait_recv()

        wait_recv(recv_nbr_ref.at[0], nbr_recv_sems.at[0])
        out_ref[...] = out_ref[...] + recv_nbr_ref[0].astype(jnp.float32)

        wait_recv(recv_dq_ref, diag_recv_sems.at[0])
        wait_recv(recv_ds_ref, diag_recv_sems.at[1])
        out_ref[...] = out_ref[...] + (
            recv_dq_ref[...].astype(jnp.float32)
            * recv_ds_ref[:, 0:1].astype(jnp.float32)
        )

        wait_recv(recv_nbr_ref.at[1], nbr_recv_sems.at[1])
        out_ref[...] = out_ref[...] + recv_nbr_ref[1].astype(jnp.float32)

        amax = jnp.max(jnp.abs(out_ref[...]))
        amax_send_ref[...] = jnp.full((8, 128), amax, jnp.float32)
        for t in range(1, N_DEV):
            tgt = lax.rem(my + t, N_DEV)
            pltpu.make_async_remote_copy(
                src_ref=amax_send_ref,
                dst_ref=amax_recv_ref.at[t - 1],
                send_sem=amax_send_sems.at[t - 1],
                recv_sem=amax_recv_sems.at[t - 1],
                device_id=(tgt,),
                device_id_type=pl.DeviceIdType.MESH,
            ).start()
        for j in range(N_DEV - 1):
            wait_recv(amax_recv_ref.at[j], amax_recv_sems.at[j])
            amax = jnp.maximum(amax, amax_recv_ref[j, 0, 0])

        scale = amax / 448.0
        q = jnp.clip(out_ref[...] / scale, -448.0, 448.0).astype(
            jnp.float8_e4m3fn
        )
        out_ref[...] = q.astype(jnp.float32) * scale

        def wait_send(src, sem):
            pltpu.make_async_remote_copy(
                src_ref=src, dst_ref=src, send_sem=sem, recv_sem=sem,
                device_id=(my,), device_id_type=pl.DeviceIdType.MESH,
            ).wait_send()

        wait_send(send_nbr_ref.at[0], nbr_send_sems.at[0])
        wait_send(send_nbr_ref.at[1], nbr_send_sems.at[1])
        wait_send(send_dq_ref, diag_send_sems.at[0])
        wait_send(send_ds_ref, diag_send_sems.at[1])
        for j in range(N_DEV - 1):
            wait_send(amax_send_ref, amax_send_sems.at[j])

    return pl.pallas_call(
        body,
        out_shape=jax.ShapeDtypeStruct((Mc, N), jnp.float32),
        in_specs=[
            pl.BlockSpec(memory_space=pltpu.MemorySpace.HBM),
            pl.BlockSpec(memory_space=pltpu.VMEM),
        ],
        out_specs=pl.BlockSpec(memory_space=pltpu.VMEM),
        scratch_shapes=[
            pltpu.VMEM((2, Mc, K), jnp.float32),
            pltpu.VMEM((2, Mc, N), jnp.bfloat16),
            pltpu.VMEM((2, Mc, N), jnp.bfloat16),
            pltpu.VMEM((Mc, N), jnp.int8),
            pltpu.VMEM((Mc, 128), jnp.bfloat16),
            pltpu.VMEM((Mc, N), jnp.int8),
            pltpu.VMEM((Mc, 128), jnp.bfloat16),
            pltpu.VMEM((8, 128), jnp.float32),
            pltpu.VMEM((N_DEV - 1, 8, 128), jnp.float32),
            pltpu.SemaphoreType.DMA((2,)),
            pltpu.SemaphoreType.DMA((2,)),
            pltpu.SemaphoreType.DMA((2,)),
            pltpu.SemaphoreType.DMA((2,)),
            pltpu.SemaphoreType.DMA((2,)),
            pltpu.SemaphoreType.DMA((N_DEV - 1,)),
            pltpu.SemaphoreType.DMA((N_DEV - 1,)),
        ],
        compiler_params=pltpu.CompilerParams(
            vmem_limit_bytes=63 * 1024 * 1024,
        ),
    )(x, w_mat.astype(jnp.bfloat16))


# device time: 101057 ns/iter; 1.4054x vs baseline; 1.1241x over previous
import jax
import jax.numpy as jnp
from jax import lax
from jax.experimental import pallas as pl
from jax.experimental.pallas import tpu as pltpu

N_DEV = 4
BS = 128


def kernel(x, w_mat):
    M, K = x.shape
    _, N = w_mat.shape
    Mc = M // N_DEV
    NB = N // BS

    def body(
        x_ref, w_ref, out_ref,
        xbuf_ref,
        send_q_ref, send_s_ref, recv_q_ref, recv_s_ref,
        amax_send_ref, amax_recv_ref,
        xcopy_sems,
        q_send_sems, q_recv_sems, s_send_sems, s_recv_sems,
        amax_send_sems, amax_recv_sems,
    ):
        my = lax.axis_index("i")

        def x_copy(c, slot):
            return pltpu.make_async_copy(
                x_ref.at[pl.ds(c * Mc, Mc), :],
                xbuf_ref.at[slot],
                xcopy_sems.at[slot],
            )

        chunks = [lax.rem(my + t, N_DEV) for t in (1, 2, 3)] + [my]
        x_copy(chunks[0], 0).start()
        x_copy(chunks[1], 1).start()

        barrier = pltpu.get_barrier_semaphore()
        for t in range(1, N_DEV):
            pl.semaphore_signal(
                barrier, inc=1,
                device_id=(lax.rem(my + t, N_DEV),),
                device_id_type=pl.DeviceIdType.MESH,
            )
        pl.semaphore_wait(barrier, N_DEV - 1)

        def dot_chunk(k):
            slot = k % 2
            x_copy(chunks[k], slot).wait()
            p = jnp.dot(
                xbuf_ref[slot].astype(jnp.bfloat16), w_ref[...],
                preferred_element_type=jnp.float32,
            )
            if k + 2 < N_DEV:
                x_copy(chunks[k + 2], slot).start()
            return p

        for k in range(3):
            p = dot_chunk(k)
            pb = p.reshape(Mc, NB, BS)
            s = (
                jnp.maximum(
                    jnp.max(jnp.abs(pb), axis=2, keepdims=True) / 127.0,
                    1e-20,
                )
            ).astype(jnp.bfloat16)
            send_s_ref[k, :, :] = s.reshape(Mc, NB)
            send_q_ref[k, :, :] = (
                jnp.clip(jnp.round(pb / s.astype(jnp.float32)), -127.0, 127.0)
                .astype(jnp.int8)
                .reshape(Mc, N)
            )
            pltpu.make_async_remote_copy(
                src_ref=send_q_ref.at[k],
                dst_ref=recv_q_ref.at[k],
                send_sem=q_send_sems.at[k],
                recv_sem=q_recv_sems.at[k],
                device_id=(chunks[k],),
                device_id_type=pl.DeviceIdType.MESH,
            ).start()
            pltpu.make_async_remote_copy(
                src_ref=send_s_ref.at[k],
                dst_ref=recv_s_ref.at[k],
                send_sem=s_send_sems.at[k],
                recv_sem=s_recv_sems.at[k],
                device_id=(chunks[k],),
                device_id_type=pl.DeviceIdType.MESH,
            ).start()

        out_ref[...] = dot_chunk(3)

        def wait_recv(dst, sem):
            pltpu.make_async_remote_copy(
                src_ref=dst, dst_ref=dst, send_sem=sem, recv_sem=sem,
                device_id=(my,), device_id_type=pl.DeviceIdType.MESH,
            ).wait_recv()

        for j in range(3):
            wait_recv(recv_q_ref.at[j], q_recv_sems.at[j])
            wait_recv(recv_s_ref.at[j], s_recv_sems.at[j])
            dq = (
                recv_q_ref[j].reshape(Mc, NB, BS).astype(jnp.float32)
                * recv_s_ref[j].astype(jnp.float32).reshape(Mc, NB, 1)
            ).reshape(Mc, N)
            out_ref[...] = out_ref[...] + dq

        amax = jnp.max(jnp.abs(out_ref[...]))
        amax_send_ref[...] = jnp.full((8, 128), amax, jnp.float32)
        for t in range(1, N_DEV):
            tgt = lax.rem(my + t, N_DEV)
            pltpu.make_async_remote_copy(
                src_ref=amax_send_ref,
                dst_ref=amax_recv_ref.at[t - 1],
                send_sem=amax_send_sems.at[t - 1],
                recv_sem=amax_recv_sems.at[t - 1],
                device_id=(tgt,),
                device_id_type=pl.DeviceIdType.MESH,
            ).start()
        for j in range(N_DEV - 1):
            wait_recv(amax_recv_ref.at[j], amax_recv_sems.at[j])
            amax = jnp.maximum(amax, amax_recv_ref[j, 0, 0])

        scale = amax / 448.0
        q = jnp.clip(out_ref[...] / scale, -448.0, 448.0).astype(
            jnp.float8_e4m3fn
        )
        out_ref[...] = q.astype(jnp.float32) * scale

        def wait_send(src, sem):
            pltpu.make_async_remote_copy(
                src_ref=src, dst_ref=src, send_sem=sem, recv_sem=sem,
                device_id=(my,), device_id_type=pl.DeviceIdType.MESH,
            ).wait_send()

        for j in range(3):
            wait_send(send_q_ref.at[j], q_send_sems.at[j])
            wait_send(send_s_ref.at[j], s_send_sems.at[j])
            wait_send(amax_send_ref, amax_send_sems.at[j])

    return pl.pallas_call(
        body,
        out_shape=jax.ShapeDtypeStruct((Mc, N), jnp.float32),
        in_specs=[
            pl.BlockSpec(memory_space=pltpu.MemorySpace.HBM),
            pl.BlockSpec(memory_space=pltpu.VMEM),
        ],
        out_specs=pl.BlockSpec(memory_space=pltpu.VMEM),
        scratch_shapes=[
            pltpu.VMEM((2, Mc, K), jnp.float32),
            pltpu.VMEM((3, Mc, N), jnp.int8),
            pltpu.VMEM((3, Mc, NB), jnp.bfloat16),
            pltpu.VMEM((3, Mc, N), jnp.int8),
            pltpu.VMEM((3, Mc, NB), jnp.bfloat16),
            pltpu.VMEM((8, 128), jnp.float32),
            pltpu.VMEM((N_DEV - 1, 8, 128), jnp.float32),
            pltpu.SemaphoreType.DMA((2,)),
            pltpu.SemaphoreType.DMA((3,)),
            pltpu.SemaphoreType.DMA((3,)),
            pltpu.SemaphoreType.DMA((3,)),
            pltpu.SemaphoreType.DMA((3,)),
            pltpu.SemaphoreType.DMA((N_DEV - 1,)),
            pltpu.SemaphoreType.DMA((N_DEV - 1,)),
        ],
        compiler_params=pltpu.CompilerParams(
            vmem_limit_bytes=63 * 1024 * 1024,
            collective_id=0,
        ),
    )(x, w_mat.astype(jnp.bfloat16))
